# Initial kernel scaffold; baseline (speedup 1.0000x reference)
import jax
import jax.numpy as jnp
from jax.experimental import pallas as pl


def kernel(user_embedding1, user_embedding2, user_embedding3, user_embedding4, user_embedding5, movie_embedding, T_u1, T_u2, T_movie, W1, b1, W2, b2, W3, b3):
    raise NotImplementedError("write your pallas kernel here")


# speedup vs baseline: 1.2798x; 1.2798x over previous
import functools

import jax
import jax.numpy as jnp
from jax import lax
from jax.experimental import pallas as pl
from jax.experimental.pallas import tpu as pltpu
from jax.experimental.pallas import tpu_sc as plsc

B = 16384
D = 200
NUM_EMB = 6

_info = plsc.get_sparse_core_info()
_NC, _NS = _info.num_cores, _info.num_subcores
_NW = _NC * _NS
_BPW = B // _NW


def _gather_kernel(t_u1, t_u2, t_movie, i1, i2, i3, i4, i5, im, out,
                   idx_v, rows_v, sem):
    wid = lax.axis_index("s") * _NC + lax.axis_index("c")
    base = wid * _BPW
    pairs = ((t_u1, i1), (t_u2, i2), (t_u2, i3), (t_u2, i4), (t_u2, i5),
             (t_movie, im))
    for j, (table, idx_hbm) in enumerate(pairs):
        pltpu.sync_copy(idx_hbm.at[pl.ds(base, _BPW)], idx_v)
        pltpu.async_copy(table.at[idx_v], rows_v, sem).wait()
        pltpu.sync_copy(rows_v, out.at[pl.ds(j * B + base, _BPW)])


_gather = functools.partial(
    pl.kernel,
    mesh=plsc.VectorSubcoreMesh(core_axis_name="c", subcore_axis_name="s"),
    compiler_params=pltpu.CompilerParams(use_tc_tiling_on_sc=False),
    out_type=jax.ShapeDtypeStruct((NUM_EMB * B, D), jnp.float32),
    scratch_types=[
        pltpu.VMEM((_BPW,), jnp.int32),
        pltpu.VMEM((_BPW, D), jnp.float32),
        pltpu.SemaphoreType.DMA,
    ],
)(_gather_kernel)


_BLK = 512


def _mlp_kernel(x_ref, w1_ref, w2_ref, w3_ref, b1_ref, b2_ref, b3_ref,
                out_ref):
    h = jnp.zeros((_BLK, 128), jnp.float32)
    for j in range(NUM_EMB):
        h = h + lax.dot_general(
            x_ref[j], w1_ref[j * D:(j + 1) * D, :],
            (((1,), (0,)), ((), ())), preferred_element_type=jnp.float32)
    h = jnp.maximum(h + b1_ref[...], 0.0)
    h = lax.dot_general(h, w2_ref[...], (((1,), (0,)), ((), ())),
                        preferred_element_type=jnp.float32)
    h = jnp.maximum(h + b2_ref[...], 0.0)
    logits = lax.dot_general(h, w3_ref[...], (((1,), (0,)), ((), ())),
                             preferred_element_type=jnp.float32)
    logits = logits + b3_ref[...]
    m = jnp.max(logits, axis=1, keepdims=True)
    e = jnp.exp(logits - m)
    out_ref[...] = e / jnp.sum(e, axis=1, keepdims=True)


def kernel(user_embedding1, user_embedding2, user_embedding3,
           user_embedding4, user_embedding5, movie_embedding,
           T_u1, T_u2, T_movie, W1, b1, W2, b2, W3, b3):
    idx = [jnp.asarray(i, jnp.int32) for i in
           (user_embedding1, user_embedding2, user_embedding3,
            user_embedding4, user_embedding5, movie_embedding)]
    rows = _gather(T_u1, T_u2, T_movie, *idx)
    xall = rows.reshape(NUM_EMB, B, D)

    w1t = W1.T
    w2t = W2.T
    w3t = W3.T
    b1r = b1.reshape(1, -1)
    b2r = b2.reshape(1, -1)
    b3r = b3.reshape(1, -1)

    grid = B // _BLK
    out = pl.pallas_call(
        _mlp_kernel,
        grid=(grid,),
        in_specs=[
            pl.BlockSpec((NUM_EMB, _BLK, D), lambda i: (0, i, 0)),
            pl.BlockSpec((NUM_EMB * D, 128), lambda i: (0, 0)),
            pl.BlockSpec((128, 64), lambda i: (0, 0)),
            pl.BlockSpec((64, 5), lambda i: (0, 0)),
            pl.BlockSpec((1, 128), lambda i: (0, 0)),
            pl.BlockSpec((1, 64), lambda i: (0, 0)),
            pl.BlockSpec((1, 5), lambda i: (0, 0)),
        ],
        out_specs=pl.BlockSpec((_BLK, 5), lambda i: (i, 0)),
        out_shape=jax.ShapeDtypeStruct((B, 5), jnp.float32),
    )(xall, w1t, w2t, w3t, b1r, b2r, b3r)
    return out


# speedup vs baseline: 3.8472x; 3.0062x over previous
import functools

import jax
import jax.numpy as jnp
from jax import lax
from jax.experimental import pallas as pl
from jax.experimental.pallas import tpu as pltpu
from jax.experimental.pallas import tpu_sc as plsc

B = 16384
D = 200
H1 = 128
NUM_EMB = 6

_info = plsc.get_sparse_core_info()
_NC, _NS = _info.num_cores, _info.num_subcores
_NW = _NC * _NS
_BPW = B // _NW
_CH = 128

_VB = 2000


def _proj_kernel(t1_ref, t2_ref, tm_ref, w_ref, *p_refs):
    srcs = (t1_ref, t2_ref, t2_ref, t2_ref, t2_ref, tm_ref)
    for j in range(NUM_EMB):
        p_refs[j][...] = lax.dot_general(
            srcs[j][...], w_ref[j * D:(j + 1) * D, :],
            (((1,), (0,)), ((), ())), preferred_element_type=jnp.float32)


def _gather_kernel(p0, p1, p2, p3, p4, p5, i0, i1, i2, i3, i4, i5, out,
                   idx_v, rows_v, sem):
    wid = lax.axis_index("s") * _NC + lax.axis_index("c")
    base = wid * _BPW
    tables = (p0, p1, p2, p3, p4, p5)
    idxs = (i0, i1, i2, i3, i4, i5)
    for j in range(NUM_EMB):
        pltpu.sync_copy(idxs[j].at[pl.ds(base, _BPW)], idx_v)
        for c in range(_BPW // _CH):
            pltpu.async_copy(
                tables[j].at[idx_v.at[pl.ds(c * _CH, _CH)]], rows_v,
                sem).wait()
            pltpu.sync_copy(
                rows_v, out.at[pl.ds(j * B + base + c * _CH, _CH)])


_gather = functools.partial(
    pl.kernel,
    mesh=plsc.VectorSubcoreMesh(core_axis_name="c", subcore_axis_name="s"),
    out_type=jax.ShapeDtypeStruct((NUM_EMB * B, H1), jnp.float32),
    scratch_types=[
        pltpu.VMEM((_BPW,), jnp.int32),
        pltpu.VMEM((_CH, H1), jnp.float32),
        pltpu.SemaphoreType.DMA,
    ],
)(_gather_kernel)


_BLK = 512


def _finish_kernel(x_ref, w2_ref, w3_ref, b1_ref, b2_ref, b3_ref, out_ref):
    h = x_ref[0] + x_ref[1] + x_ref[2] + x_ref[3] + x_ref[4] + x_ref[5]
    h = jnp.maximum(h + b1_ref[...], 0.0)
    h = lax.dot_general(h, w2_ref[...], (((1,), (0,)), ((), ())),
                        preferred_element_type=jnp.float32)
    h = jnp.maximum(h + b2_ref[...], 0.0)
    logits = lax.dot_general(h, w3_ref[...], (((1,), (0,)), ((), ())),
                             preferred_element_type=jnp.float32)
    logits = logits + b3_ref[...]
    m = jnp.max(logits, axis=1, keepdims=True)
    e = jnp.exp(logits - m)
    out_ref[...] = e / jnp.sum(e, axis=1, keepdims=True)


def kernel(user_embedding1, user_embedding2, user_embedding3,
           user_embedding4, user_embedding5, movie_embedding,
           T_u1, T_u2, T_movie, W1, b1, W2, b2, W3, b3):
    idx = [jnp.asarray(i, jnp.int32) for i in
           (user_embedding1, user_embedding2, user_embedding3,
            user_embedding4, user_embedding5, movie_embedding)]
    w1t = W1.T

    vocab = T_u1.shape[0]
    proj = pl.pallas_call(
        _proj_kernel,
        grid=(vocab // _VB,),
        in_specs=[
            pl.BlockSpec((_VB, D), lambda i: (i, 0)),
            pl.BlockSpec((_VB, D), lambda i: (i, 0)),
            pl.BlockSpec((_VB, D), lambda i: (i, 0)),
            pl.BlockSpec((NUM_EMB * D, H1), lambda i: (0, 0)),
        ],
        out_specs=[pl.BlockSpec((_VB, H1), lambda i: (i, 0))
                   for _ in range(NUM_EMB)],
        out_shape=[jax.ShapeDtypeStruct((vocab, H1), jnp.float32)
                   for _ in range(NUM_EMB)],
    )(T_u1, T_u2, T_movie, w1t)

    rows = _gather(*proj, *idx)
    xall = rows.reshape(NUM_EMB, B, H1)

    grid = B // _BLK
    out = pl.pallas_call(
        _finish_kernel,
        grid=(grid,),
        in_specs=[
            pl.BlockSpec((NUM_EMB, _BLK, H1), lambda i: (0, i, 0)),
            pl.BlockSpec((H1, 64), lambda i: (0, 0)),
            pl.BlockSpec((64, 5), lambda i: (0, 0)),
            pl.BlockSpec((1, H1), lambda i: (0, 0)),
            pl.BlockSpec((1, 64), lambda i: (0, 0)),
            pl.BlockSpec((1, 5), lambda i: (0, 0)),
        ],
        out_specs=pl.BlockSpec((_BLK, 5), lambda i: (i, 0)),
        out_shape=jax.ShapeDtypeStruct((B, 5), jnp.float32),
    )(xall, W2.T, W3.T, b1.reshape(1, -1), b2.reshape(1, -1),
      b3.reshape(1, -1))
    return out
